# Initial kernel scaffold; baseline (speedup 1.0000x reference)
import jax
import jax.numpy as jnp
from jax.experimental import pallas as pl


def kernel(x, pos, batch, params):
    raise NotImplementedError("write your pallas kernel here")


# speedup vs baseline: 5.4151x; 5.4151x over previous
import functools
import jax
import jax.numpy as jnp
from jax.experimental import pallas as pl
from jax.experimental.pallas import tpu as pltpu

EPS = 1e-5
K = 16


def _knn_body(q_ref, p_ref, o_ref, *, n, k, exclude_self, blk_r):
    r0 = pl.program_id(0) * blk_r
    q = q_ref[...]
    p = p_ref[...]
    qq = jnp.sum(q * q, axis=1)[:, None]
    pp = jnp.sum(p * p, axis=1)[None, :]
    qp = jax.lax.dot_general(q, p, (((1,), (1,)), ((), ())),
                             preferred_element_type=jnp.float32)
    d = qq + pp - 2.0 * qp
    col = jax.lax.broadcasted_iota(jnp.int32, (blk_r, n), 1)
    if exclude_self:
        row = jax.lax.broadcasted_iota(jnp.int32, (blk_r, n), 0) + r0
        d = jnp.where(col == row, d + 1e10, d)
    for j in range(k):
        mv = jnp.min(d, axis=1, keepdims=True)
        idxj = jnp.min(jnp.where(d == mv, col, n), axis=1)
        o_ref[:, j] = idxj
        d = jnp.where(col == idxj[:, None], jnp.float32(jnp.inf), d)


def _knn_idx(query, pos, k, exclude_self):
    m = query.shape[0]
    n = pos.shape[0]
    blk_r = min(m, 256)
    grid = m // blk_r
    return pl.pallas_call(
        functools.partial(_knn_body, n=n, k=k, exclude_self=exclude_self,
                          blk_r=blk_r),
        grid=(grid,),
        in_specs=[
            pl.BlockSpec((blk_r, 3), lambda i: (i, 0)),
            pl.BlockSpec((n, 3), lambda i: (0, 0)),
        ],
        out_specs=pl.BlockSpec((blk_r, k), lambda i: (i, 0)),
        out_shape=jax.ShapeDtypeStruct((m, k), jnp.int32),
    )(query, pos)


def _fps_body(xyz_ref, o_ref, d_ref, *, n, m, S):
    X = xyz_ref[0]
    Y = xyz_ref[1]
    Z = xyz_ref[2]
    fiota = (jax.lax.broadcasted_iota(jnp.int32, (S, 128), 0) * 128
             + jax.lax.broadcasted_iota(jnp.int32, (S, 128), 1))
    miota = jax.lax.broadcasted_iota(jnp.int32, (1, m), 1)

    x0 = jnp.sum(jnp.where(fiota == 0, X, 0.0))
    y0 = jnp.sum(jnp.where(fiota == 0, Y, 0.0))
    z0 = jnp.sum(jnp.where(fiota == 0, Z, 0.0))
    dx = X - x0
    dy = Y - y0
    dz = Z - z0
    d_ref[...] = dx * dx + dy * dy + dz * dz
    o_ref[...] = jnp.zeros((1, m), jnp.int32)

    def body(i, _):
        d = d_ref[...]
        mval = jnp.max(d)
        nxt = jnp.min(jnp.where(d == mval, fiota, n))
        x = jnp.sum(jnp.where(fiota == nxt, X, 0.0))
        y = jnp.sum(jnp.where(fiota == nxt, Y, 0.0))
        z = jnp.sum(jnp.where(fiota == nxt, Z, 0.0))
        ddx = X - x
        ddy = Y - y
        ddz = Z - z
        dn = ddx * ddx + ddy * ddy + ddz * ddz
        d_ref[...] = jnp.minimum(d, dn)
        o_ref[...] = jnp.where(miota == i, nxt, o_ref[...])
        return 0

    jax.lax.fori_loop(1, m, body, 0)


def _fps(pos, m):
    n = pos.shape[0]
    S = n // 128
    xyz = pos.T.reshape(3, S, 128)
    out = pl.pallas_call(
        functools.partial(_fps_body, n=n, m=m, S=S),
        scratch_shapes=[pltpu.VMEM((S, 128), jnp.float32)],
        out_shape=jax.ShapeDtypeStruct((1, m), jnp.int32),
    )(xyz)
    return out[0]


def _apply_mlp_bn(p, x):
    h = x @ p["W"] + p["b"]
    h = p["gamma"] * h / jnp.sqrt(1.0 + EPS) + p["beta"]
    return jax.nn.relu(h)


def _mlp2(p, x):
    h = jax.nn.relu(x @ p["l1"]["W"] + p["l1"]["b"])
    return jax.nn.relu(h @ p["l2"]["W"] + p["l2"]["b"])


def _pt_conv(p, x, pos, idx):
    n = x.shape[0]
    h = x @ p["W_lin"]
    a_src = x @ p["W_src"]
    a_dst = x @ p["W_dst"]
    pv = pos[:, None, :] - pos[idx]
    delta = _mlp2(p["pos_nn"], pv)
    ain = a_dst[:, None, :] - a_src[idx] + delta
    alpha = _mlp2(p["attn_nn"], ain)
    mx = jnp.max(alpha, axis=1, keepdims=True)
    e = jnp.exp(alpha - mx)
    s = jnp.sum(e, axis=1, keepdims=True)
    alpha = e / (s + 1e-16)
    msg = alpha * (h[idx] + delta)
    return jnp.sum(msg, axis=1)


def _tblock(p, x, pos, idx):
    x = jax.nn.relu(x @ p["lin_in"]["W"] + p["lin_in"]["b"])
    x = _pt_conv(p, x, pos, idx)
    return jax.nn.relu(x @ p["lin_out"]["W"] + p["lin_out"]["b"])


def kernel(x, pos, batch, params):
    x = _apply_mlp_bn(params["mlp_input"], x)
    idx = _knn_idx(pos, pos, K, True)
    x = _tblock(params["t_in"], x, pos, idx)
    for i in range(4):
        n_sub = x.shape[0] // 4
        idc = _fps(pos, n_sub)
        x_m = _apply_mlp_bn(params["td"][i], x)
        nbr = _knn_idx(pos[idc], pos, K, False)
        x = jnp.max(x_m[nbr], axis=1)
        pos = pos[idc]
        idx = _knn_idx(pos, pos, K, True)
        x = _tblock(params["tb"][i], x, pos, idx)
    pooled = jnp.sum(x, axis=0, keepdims=True) / x.shape[0]
    out = jax.nn.relu(pooled @ params["out1"]["W"] + params["out1"]["b"])
    out = out @ params["out2"]["W"] + params["out2"]["b"]
    return out


# speedup vs baseline: 7.1378x; 1.3181x over previous
import functools
import jax
import jax.numpy as jnp
from jax.experimental import pallas as pl
from jax.experimental.pallas import tpu as pltpu

EPS = 1e-5
K = 16


def _pad16(v):
    return (v + 15) // 16 * 16


def _knn_body(q_ref, p_ref, o_ref, *, n, k, exclude_self, blk_r):
    r0 = pl.program_id(0) * blk_r
    q = q_ref[...]
    p = p_ref[...]
    qq = jnp.sum(q * q, axis=1)[:, None]
    pp = jnp.sum(p * p, axis=1)[None, :]
    qp = jax.lax.dot_general(q, p, (((1,), (1,)), ((), ())),
                             preferred_element_type=jnp.float32)
    d = qq + pp - 2.0 * qp
    col = jax.lax.broadcasted_iota(jnp.int32, (blk_r, n), 1)
    if exclude_self:
        row = jax.lax.broadcasted_iota(jnp.int32, (blk_r, n), 0) + r0
        d = jnp.where(col == row, d + 1e10, d)
    for j in range(k):
        mv = jnp.min(d, axis=1, keepdims=True)
        idxj = jnp.min(jnp.where(d == mv, col, n), axis=1)
        o_ref[:, j] = idxj
        d = jnp.where(col == idxj[:, None], jnp.float32(jnp.inf), d)


def _knn_idx(query, pos, k, exclude_self):
    m = query.shape[0]
    n = pos.shape[0]
    blk_r = min(m, 256)
    grid = m // blk_r
    return pl.pallas_call(
        functools.partial(_knn_body, n=n, k=k, exclude_self=exclude_self,
                          blk_r=blk_r),
        grid=(grid,),
        in_specs=[
            pl.BlockSpec((blk_r, 3), lambda i: (i, 0)),
            pl.BlockSpec((n, 3), lambda i: (0, 0)),
        ],
        out_specs=pl.BlockSpec((blk_r, k), lambda i: (i, 0)),
        out_shape=jax.ShapeDtypeStruct((m, k), jnp.int32),
    )(query, pos)


def _fps_body(xyz_ref, o_ref, d_ref, *, n, m, S):
    X = xyz_ref[0]
    Y = xyz_ref[1]
    Z = xyz_ref[2]
    fiota = (jax.lax.broadcasted_iota(jnp.int32, (S, 128), 0) * 128
             + jax.lax.broadcasted_iota(jnp.int32, (S, 128), 1))
    miota = jax.lax.broadcasted_iota(jnp.int32, (1, m), 1)

    x0 = jnp.sum(jnp.where(fiota == 0, X, 0.0))
    y0 = jnp.sum(jnp.where(fiota == 0, Y, 0.0))
    z0 = jnp.sum(jnp.where(fiota == 0, Z, 0.0))
    dx = X - x0
    dy = Y - y0
    dz = Z - z0
    d_ref[...] = dx * dx + dy * dy + dz * dz
    o_ref[...] = jnp.zeros((1, m), jnp.int32)

    def body(i, _):
        d = d_ref[...]
        mval = jnp.max(d)
        nxt = jnp.min(jnp.where(d == mval, fiota, n))
        x = jnp.sum(jnp.where(fiota == nxt, X, 0.0))
        y = jnp.sum(jnp.where(fiota == nxt, Y, 0.0))
        z = jnp.sum(jnp.where(fiota == nxt, Z, 0.0))
        ddx = X - x
        ddy = Y - y
        ddz = Z - z
        dn = ddx * ddx + ddy * ddy + ddz * ddz
        d_ref[...] = jnp.minimum(d, dn)
        o_ref[...] = jnp.where(miota == i, nxt, o_ref[...])
        return 0

    jax.lax.fori_loop(1, m, body, 0)


def _fps(pos, m):
    n = pos.shape[0]
    S = n // 128
    xyz = pos.T.reshape(3, S, 128)
    out = pl.pallas_call(
        functools.partial(_fps_body, n=n, m=m, S=S),
        scratch_shapes=[pltpu.VMEM((S, 128), jnp.float32)],
        out_shape=jax.ShapeDtypeStruct((1, m), jnp.int32),
    )(xyz)
    return out[0]


def _bn(gamma, beta, h):
    return jax.nn.relu(gamma * h / jnp.sqrt(1.0 + EPS) + beta)


def _pre_body(*refs, d, P, has_entry):
    if has_entry:
        (x_ref, pos_ref, wi_ref, bi_ref, wsrc_ref, wlin_ref, wdst_ref,
         g_ref, be_ref, w0_ref, b0_ref, t_ref, adst_ref) = refs
    else:
        (x_ref, pos_ref, wi_ref, bi_ref, wsrc_ref, wlin_ref, wdst_ref,
         t_ref, adst_ref) = refs
    x = x_ref[...]
    if has_entry:
        x = _bn(g_ref[...], be_ref[...], x @ w0_ref[...] + b0_ref[...])
    if x.ndim == 3:
        x = jnp.max(x, axis=1)
    xi = jax.nn.relu(x @ wi_ref[...] + bi_ref[...])
    pad = jnp.zeros((x.shape[0], P - (2 * d + 3)), jnp.float32)
    t_ref[...] = jnp.concatenate(
        [xi @ wsrc_ref[...], xi @ wlin_ref[...], pos_ref[...], pad], axis=1)
    adst_ref[...] = xi @ wdst_ref[...]


def _tblock_pre(p, x, pos, entry=None):
    n = pos.shape[0]
    d = p["W_lin"].shape[0]
    P = _pad16(2 * d + 3)
    blk = min(n, 1024)
    grid = n // blk
    if x.ndim == 3:
        x_spec = pl.BlockSpec((blk, x.shape[1], x.shape[2]),
                              lambda i: (i, 0, 0))
    else:
        x_spec = pl.BlockSpec((blk, x.shape[1]), lambda i: (i, 0))
    full = lambda a: pl.BlockSpec(a.shape, lambda i: tuple(0 for _ in a.shape))
    args = [x, pos, p["lin_in"]["W"], p["lin_in"]["b"][None, :],
            p["W_src"], p["W_lin"], p["W_dst"]]
    specs = [x_spec, pl.BlockSpec((blk, 3), lambda i: (i, 0))] + [
        full(a) for a in args[2:]]
    if entry is not None:
        eargs = [entry["gamma"][None, :], entry["beta"][None, :],
                 entry["W"], entry["b"][None, :]]
        args += eargs
        specs += [full(a) for a in eargs]
    T, adst = pl.pallas_call(
        functools.partial(_pre_body, d=d, P=P, has_entry=entry is not None),
        grid=(grid,),
        in_specs=specs,
        out_specs=[pl.BlockSpec((blk, P), lambda i: (i, 0)),
                   pl.BlockSpec((blk, d), lambda i: (i, 0))],
        out_shape=[jax.ShapeDtypeStruct((n, P), jnp.float32),
                   jax.ShapeDtypeStruct((n, d), jnp.float32)],
    )(*args)
    return T, adst


def _conv_body(g_ref, adst_ref, pos_ref,
               pw1_ref, pb1_ref, pw2_ref, pb2_ref,
               aw1_ref, ab1_ref, aw2_ref, ab2_ref,
               wo_ref, bo_ref, wn_ref, bn_ref, gn_ref, ben_ref,
               o_ref, *, d, P, blk, head, n_total):
    e = blk * K
    G = g_ref[...].reshape(e, P)
    a_src = G[:, 0:d]
    h = G[:, d:2 * d]
    psrc = G[:, 2 * d:2 * d + 3]
    pos = pos_ref[...]
    pdst = jnp.broadcast_to(pos[:, None, :], (blk, K, 3)).reshape(e, 3)
    pv = pdst - psrc
    t1 = jax.nn.relu(pv @ pw1_ref[...] + pb1_ref[...])
    delta = jax.nn.relu(t1 @ pw2_ref[...] + pb2_ref[...])
    adst = adst_ref[...]
    ain = (jnp.broadcast_to(adst[:, None, :], (blk, K, d)).reshape(e, d)
           - a_src + delta)
    t2 = jax.nn.relu(ain @ aw1_ref[...] + ab1_ref[...])
    alpha = jax.nn.relu(t2 @ aw2_ref[...] + ab2_ref[...])
    a3 = alpha.reshape(blk, K, d)
    mx = jnp.max(a3, axis=1, keepdims=True)
    ex = jnp.exp(a3 - mx)
    s = jnp.sum(ex, axis=1, keepdims=True)
    al = ex / (s + 1e-16)
    msg = al * (h + delta).reshape(blk, K, d)
    conv = jnp.sum(msg, axis=1)
    x = jax.nn.relu(conv @ wo_ref[...] + bo_ref[...])
    if head:
        pooled = jnp.sum(x, axis=0, keepdims=True) / n_total
        o1 = jax.nn.relu(pooled @ wn_ref[...] + bn_ref[...])
        o_ref[...] = o1 @ gn_ref[...] + ben_ref[...]
    else:
        h2 = x @ wn_ref[...] + bn_ref[...]
        o_ref[...] = _bn(gn_ref[...], ben_ref[...], h2)


def _tblock_conv(p, G, adst, pos, nxt):
    n, _, P = G.shape
    d = p["W_lin"].shape[0]
    blk = min(n, 256)
    grid = n // blk
    head = nxt[0] == "head"
    if head:
        wn, bn_, gn, ben = (nxt[1]["W"], nxt[1]["b"][None, :],
                            nxt[2]["W"], nxt[2]["b"][None, :])
        out_shape = jax.ShapeDtypeStruct((1, 10), jnp.float32)
        out_spec = pl.BlockSpec((1, 10), lambda i: (0, 0))
    else:
        td = nxt[1]
        wn, bn_, gn, ben = (td["W"], td["b"][None, :],
                            td["gamma"][None, :], td["beta"][None, :])
        d_next = td["W"].shape[1]
        out_shape = jax.ShapeDtypeStruct((n, d_next), jnp.float32)
        out_spec = pl.BlockSpec((blk, d_next), lambda i: (i, 0))
    full = lambda a: pl.BlockSpec(a.shape, lambda i: tuple(0 for _ in a.shape))
    args = [G, adst, pos,
            p["pos_nn"]["l1"]["W"], p["pos_nn"]["l1"]["b"][None, :],
            p["pos_nn"]["l2"]["W"], p["pos_nn"]["l2"]["b"][None, :],
            p["attn_nn"]["l1"]["W"], p["attn_nn"]["l1"]["b"][None, :],
            p["attn_nn"]["l2"]["W"], p["attn_nn"]["l2"]["b"][None, :],
            p["lin_out"]["W"], p["lin_out"]["b"][None, :],
            wn, bn_, gn, ben]
    specs = [pl.BlockSpec((blk, K, P), lambda i: (i, 0, 0)),
             pl.BlockSpec((blk, d), lambda i: (i, 0)),
             pl.BlockSpec((blk, 3), lambda i: (i, 0))] + [
        full(a) for a in args[3:]]
    return pl.pallas_call(
        functools.partial(_conv_body, d=d, P=P, blk=blk, head=head,
                          n_total=n),
        grid=(grid,),
        in_specs=specs,
        out_specs=out_spec,
        out_shape=out_shape,
    )(*args)


def kernel(x, pos, batch, params):
    idx = _knn_idx(pos, pos, K, True)
    T, adst = _tblock_pre(params["t_in"], x, pos, entry=params["mlp_input"])
    G = T[idx]
    x = _tblock_conv(params["t_in"], G, adst, pos,
                     ("mlp", params["td"][0]))
    for i in range(4):
        n = pos.shape[0]
        n_sub = n // 4
        idc = _fps(pos, n_sub)
        nbr = _knn_idx(pos[idc], pos, K, False)
        xg = x[nbr]
        pos = pos[idc]
        idx = _knn_idx(pos, pos, K, True)
        p = params["tb"][i]
        T, adst = _tblock_pre(p, xg, pos)
        G = T[idx]
        nxt = (("mlp", params["td"][i + 1]) if i < 3
               else ("head", params["out1"], params["out2"]))
        x = _tblock_conv(p, G, adst, pos, nxt)
    return x


# speedup vs baseline: 8.6765x; 1.2156x over previous
import functools
import jax
import jax.numpy as jnp
from jax import lax
from jax.experimental import pallas as pl
from jax.experimental.pallas import tpu as pltpu
from jax.experimental.pallas import tpu_sc as plsc

EPS = 1e-5
K = 16


def _pad128(v):
    return (v + 127) // 128 * 128


def _knn_body(q_ref, p_ref, o_ref, *, n, k, exclude_self, blk_r):
    r0 = pl.program_id(0) * blk_r
    q = q_ref[...]
    p = p_ref[...]
    qq = jnp.sum(q * q, axis=1)[:, None]
    pp = jnp.sum(p * p, axis=1)[None, :]
    qp = jax.lax.dot_general(q, p, (((1,), (1,)), ((), ())),
                             preferred_element_type=jnp.float32)
    d = qq + pp - 2.0 * qp
    col = jax.lax.broadcasted_iota(jnp.int32, (blk_r, n), 1)
    if exclude_self:
        row = jax.lax.broadcasted_iota(jnp.int32, (blk_r, n), 0) + r0
        d = jnp.where(col == row, d + 1e10, d)
    for j in range(k):
        mv = jnp.min(d, axis=1, keepdims=True)
        idxj = jnp.min(jnp.where(d == mv, col, n), axis=1)
        o_ref[:, j] = idxj
        d = jnp.where(col == idxj[:, None], jnp.float32(jnp.inf), d)


def _knn_idx(query, pos, k, exclude_self):
    m = query.shape[0]
    n = pos.shape[0]
    blk_r = min(m, 256)
    grid = m // blk_r
    return pl.pallas_call(
        functools.partial(_knn_body, n=n, k=k, exclude_self=exclude_self,
                          blk_r=blk_r),
        grid=(grid,),
        in_specs=[
            pl.BlockSpec((blk_r, 3), lambda i: (i, 0)),
            pl.BlockSpec((n, 3), lambda i: (0, 0)),
        ],
        out_specs=pl.BlockSpec((blk_r, k), lambda i: (i, 0)),
        out_shape=jax.ShapeDtypeStruct((m, k), jnp.int32),
    )(query, pos)


def _fps_body(xyz_ref, o_ref, d_ref, *, n, m, S):
    X = xyz_ref[0]
    Y = xyz_ref[1]
    Z = xyz_ref[2]
    fiota = (jax.lax.broadcasted_iota(jnp.int32, (S, 128), 0) * 128
             + jax.lax.broadcasted_iota(jnp.int32, (S, 128), 1))
    miota = jax.lax.broadcasted_iota(jnp.int32, (1, m), 1)

    x0 = jnp.sum(jnp.where(fiota == 0, X, 0.0))
    y0 = jnp.sum(jnp.where(fiota == 0, Y, 0.0))
    z0 = jnp.sum(jnp.where(fiota == 0, Z, 0.0))
    dx = X - x0
    dy = Y - y0
    dz = Z - z0
    d_ref[...] = dx * dx + dy * dy + dz * dz
    o_ref[...] = jnp.zeros((1, m), jnp.int32)

    def body(i, _):
        d = d_ref[...]
        mval = jnp.max(d)
        nxt = jnp.min(jnp.where(d == mval, fiota, n))
        x = jnp.sum(jnp.where(fiota == nxt, X, 0.0))
        y = jnp.sum(jnp.where(fiota == nxt, Y, 0.0))
        z = jnp.sum(jnp.where(fiota == nxt, Z, 0.0))
        ddx = X - x
        ddy = Y - y
        ddz = Z - z
        dn = ddx * ddx + ddy * ddy + ddz * ddz
        d_ref[...] = jnp.minimum(d, dn)
        o_ref[...] = jnp.where(miota == i, nxt, o_ref[...])
        return 0

    jax.lax.fori_loop(1, m, body, 0)


def _fps(pos, m):
    n = pos.shape[0]
    S = n // 128
    xyz = pos.T.reshape(3, S, 128)
    out = pl.pallas_call(
        functools.partial(_fps_body, n=n, m=m, S=S),
        scratch_shapes=[pltpu.VMEM((S, 128), jnp.float32)],
        out_shape=jax.ShapeDtypeStruct((1, m), jnp.int32),
    )(xyz)
    return out[0]


def _sc_gather(table, idx):
    V, D = table.shape
    B = idx.shape[0]
    NW = 32
    b_per_w = B // NW
    chunk = min(b_per_w, 128)
    while 2 * chunk * D * 4 > 400000:
        chunk //= 2
    nchunks = b_per_w // chunk
    mesh = plsc.VectorSubcoreMesh(core_axis_name="c", subcore_axis_name="s")

    @functools.partial(
        pl.kernel, mesh=mesh,
        out_type=jax.ShapeDtypeStruct((B, D), jnp.float32),
        scratch_types=[
            pltpu.VMEM((b_per_w,), jnp.int32),
            pltpu.VMEM((2, chunk, D), jnp.float32),
            pltpu.SemaphoreType.DMA,
            pltpu.SemaphoreType.DMA,
        ],
    )
    def k(table_hbm, idx_hbm, out_hbm, idx_v, rows_v, sem0, sem1):
        wid = lax.axis_index("s") * 2 + lax.axis_index("c")
        base = wid * b_per_w
        sems = (sem0, sem1)
        pltpu.sync_copy(idx_hbm.at[pl.ds(base, b_per_w)], idx_v)
        cps = [None, None]
        cps[0] = pltpu.async_copy(
            table_hbm.at[idx_v.at[pl.ds(0, chunk)]], rows_v.at[0], sem0)
        for c in range(nchunks):
            cb = c % 2
            nb = (c + 1) % 2
            if c + 1 < nchunks:
                cps[nb] = pltpu.async_copy(
                    table_hbm.at[idx_v.at[pl.ds((c + 1) * chunk, chunk)]],
                    rows_v.at[nb], sems[nb])
            cps[cb].wait()
            pltpu.sync_copy(rows_v.at[cb],
                            out_hbm.at[pl.ds(base + c * chunk, chunk)])

    return k(table, idx)


def _bn(gamma, beta, h):
    return jax.nn.relu(gamma * h / jnp.sqrt(1.0 + EPS) + beta)


def _pre_body(*refs, d, P, has_entry):
    if has_entry:
        (x_ref, pos_ref, wi_ref, bi_ref, wsrc_ref, wlin_ref, wdst_ref,
         g_ref, be_ref, w0_ref, b0_ref, t_ref, adst_ref) = refs
    else:
        (x_ref, pos_ref, wi_ref, bi_ref, wsrc_ref, wlin_ref, wdst_ref,
         t_ref, adst_ref) = refs
    x = x_ref[...]
    if has_entry:
        x = _bn(g_ref[...], be_ref[...], x @ w0_ref[...] + b0_ref[...])
    if x.ndim == 3:
        x = jnp.max(x, axis=1)
    d_in = wi_ref.shape[0]
    if x.shape[1] > d_in:
        x = x[:, :d_in]
    xi = jax.nn.relu(x @ wi_ref[...] + bi_ref[...])
    pad = jnp.zeros((x.shape[0], P - (2 * d + 3)), jnp.float32)
    t_ref[...] = jnp.concatenate(
        [xi @ wsrc_ref[...], xi @ wlin_ref[...], pos_ref[...], pad], axis=1)
    adst_ref[...] = xi @ wdst_ref[...]


def _tblock_pre(p, x, pos, entry=None):
    n = pos.shape[0]
    d = p["W_lin"].shape[0]
    P = _pad128(2 * d + 3)
    blk = min(n, 1024)
    grid = n // blk
    if x.ndim == 3:
        x_spec = pl.BlockSpec((blk, x.shape[1], x.shape[2]),
                              lambda i: (i, 0, 0))
    else:
        x_spec = pl.BlockSpec((blk, x.shape[1]), lambda i: (i, 0))
    full = lambda a: pl.BlockSpec(a.shape, lambda i: tuple(0 for _ in a.shape))
    args = [x, pos, p["lin_in"]["W"], p["lin_in"]["b"][None, :],
            p["W_src"], p["W_lin"], p["W_dst"]]
    specs = [x_spec, pl.BlockSpec((blk, 3), lambda i: (i, 0))] + [
        full(a) for a in args[2:]]
    if entry is not None:
        eargs = [entry["gamma"][None, :], entry["beta"][None, :],
                 entry["W"], entry["b"][None, :]]
        args += eargs
        specs += [full(a) for a in eargs]
    T, adst = pl.pallas_call(
        functools.partial(_pre_body, d=d, P=P, has_entry=entry is not None),
        grid=(grid,),
        in_specs=specs,
        out_specs=[pl.BlockSpec((blk, P), lambda i: (i, 0)),
                   pl.BlockSpec((blk, d), lambda i: (i, 0))],
        out_shape=[jax.ShapeDtypeStruct((n, P), jnp.float32),
                   jax.ShapeDtypeStruct((n, d), jnp.float32)],
    )(*args)
    return T, adst


def _conv_body(g_ref, adst_ref, pos_ref,
               pw1_ref, pb1_ref, pw2_ref, pb2_ref,
               aw1_ref, ab1_ref, aw2_ref, ab2_ref,
               wo_ref, bo_ref, wn_ref, bn_ref, gn_ref, ben_ref,
               o_ref, *, d, P, blk, head, n_total):
    e = blk * K
    G = g_ref[...].reshape(e, P)
    a_src = G[:, 0:d]
    h = G[:, d:2 * d]
    psrc = G[:, 2 * d:2 * d + 3]
    pos = pos_ref[...]
    pdst = jnp.broadcast_to(pos[:, None, :], (blk, K, 3)).reshape(e, 3)
    pv = pdst - psrc
    t1 = jax.nn.relu(pv @ pw1_ref[...] + pb1_ref[...])
    delta = jax.nn.relu(t1 @ pw2_ref[...] + pb2_ref[...])
    adst = adst_ref[...]
    ain = (jnp.broadcast_to(adst[:, None, :], (blk, K, d)).reshape(e, d)
           - a_src + delta)
    t2 = jax.nn.relu(ain @ aw1_ref[...] + ab1_ref[...])
    alpha = jax.nn.relu(t2 @ aw2_ref[...] + ab2_ref[...])
    a3 = alpha.reshape(blk, K, d)
    mx = jnp.max(a3, axis=1, keepdims=True)
    ex = jnp.exp(a3 - mx)
    s = jnp.sum(ex, axis=1, keepdims=True)
    al = ex / (s + 1e-16)
    msg = al * (h + delta).reshape(blk, K, d)
    conv = jnp.sum(msg, axis=1)
    x = jax.nn.relu(conv @ wo_ref[...] + bo_ref[...])
    if head:
        pooled = jnp.sum(x, axis=0, keepdims=True) / n_total
        o1 = jax.nn.relu(pooled @ wn_ref[...] + bn_ref[...])
        o_ref[...] = o1 @ gn_ref[...] + ben_ref[...]
    else:
        h2 = x @ wn_ref[...] + bn_ref[...]
        xm = _bn(gn_ref[...], ben_ref[...], h2)
        dp = o_ref.shape[1]
        if dp > xm.shape[1]:
            xm = jnp.concatenate(
                [xm, jnp.zeros((xm.shape[0], dp - xm.shape[1]), jnp.float32)],
                axis=1)
        o_ref[...] = xm


def _tblock_conv(p, G, adst, pos, nxt):
    n, _, P = G.shape
    d = p["W_lin"].shape[0]
    blk = min(n, 256)
    grid = n // blk
    head = nxt[0] == "head"
    if head:
        wn, bn_, gn, ben = (nxt[1]["W"], nxt[1]["b"][None, :],
                            nxt[2]["W"], nxt[2]["b"][None, :])
        out_shape = jax.ShapeDtypeStruct((1, 10), jnp.float32)
        out_spec = pl.BlockSpec((1, 10), lambda i: (0, 0))
    else:
        td = nxt[1]
        wn, bn_, gn, ben = (td["W"], td["b"][None, :],
                            td["gamma"][None, :], td["beta"][None, :])
        d_next = _pad128(td["W"].shape[1])
        out_shape = jax.ShapeDtypeStruct((n, d_next), jnp.float32)
        out_spec = pl.BlockSpec((blk, d_next), lambda i: (i, 0))
    full = lambda a: pl.BlockSpec(a.shape, lambda i: tuple(0 for _ in a.shape))
    args = [G, adst, pos,
            p["pos_nn"]["l1"]["W"], p["pos_nn"]["l1"]["b"][None, :],
            p["pos_nn"]["l2"]["W"], p["pos_nn"]["l2"]["b"][None, :],
            p["attn_nn"]["l1"]["W"], p["attn_nn"]["l1"]["b"][None, :],
            p["attn_nn"]["l2"]["W"], p["attn_nn"]["l2"]["b"][None, :],
            p["lin_out"]["W"], p["lin_out"]["b"][None, :],
            wn, bn_, gn, ben]
    specs = [pl.BlockSpec((blk, K, P), lambda i: (i, 0, 0)),
             pl.BlockSpec((blk, d), lambda i: (i, 0)),
             pl.BlockSpec((blk, 3), lambda i: (i, 0))] + [
        full(a) for a in args[3:]]
    return pl.pallas_call(
        functools.partial(_conv_body, d=d, P=P, blk=blk, head=head,
                          n_total=n),
        grid=(grid,),
        in_specs=specs,
        out_specs=out_spec,
        out_shape=out_shape,
    )(*args)


def kernel(x, pos, batch, params):
    idx = _knn_idx(pos, pos, K, True)
    T, adst = _tblock_pre(params["t_in"], x, pos, entry=params["mlp_input"])
    n = pos.shape[0]
    G = _sc_gather(T, idx.reshape(-1)).reshape(n, K, T.shape[1])
    x = _tblock_conv(params["t_in"], G, adst, pos,
                     ("mlp", params["td"][0]))
    for i in range(4):
        n = pos.shape[0]
        n_sub = n // 4
        idc = _fps(pos, n_sub)
        nbr = _knn_idx(pos[idc], pos, K, False)
        xg = _sc_gather(x, nbr.reshape(-1)).reshape(n_sub, K, x.shape[1])
        pos = pos[idc]
        idx = _knn_idx(pos, pos, K, True)
        p = params["tb"][i]
        T, adst = _tblock_pre(p, xg, pos)
        G = _sc_gather(T, idx.reshape(-1)).reshape(n_sub, K, T.shape[1])
        nxt = (("mlp", params["td"][i + 1]) if i < 3
               else ("head", params["out1"], params["out2"]))
        x = _tblock_conv(p, G, adst, pos, nxt)
    return x


# speedup vs baseline: 8.6801x; 1.0004x over previous
import functools
import jax
import jax.numpy as jnp
from jax import lax
from jax.experimental import pallas as pl
from jax.experimental.pallas import tpu as pltpu
from jax.experimental.pallas import tpu_sc as plsc

EPS = 1e-5
K = 16


def _pad128(v):
    return (v + 127) // 128 * 128


def _knn_body(q_ref, p_ref, o_ref, *, n, k, exclude_self, blk_r):
    r0 = pl.program_id(0) * blk_r
    q = q_ref[...]
    p = p_ref[...]
    qq = jnp.sum(q * q, axis=1)[:, None]
    pp = jnp.sum(p * p, axis=1)[None, :]
    qp = jax.lax.dot_general(q, p, (((1,), (1,)), ((), ())),
                             preferred_element_type=jnp.float32)
    d = qq + pp - 2.0 * qp
    col = jax.lax.broadcasted_iota(jnp.int32, (blk_r, n), 1)
    if exclude_self:
        row = jax.lax.broadcasted_iota(jnp.int32, (blk_r, n), 0) + r0
        d = jnp.where(col == row, d + 1e10, d)
    for j in range(k):
        mv = jnp.min(d, axis=1, keepdims=True)
        idxj = jnp.min(jnp.where(d == mv, col, n), axis=1)
        o_ref[:, j] = idxj
        d = jnp.where(col == idxj[:, None], jnp.float32(jnp.inf), d)


def _knn_idx(query, pos, k, exclude_self):
    m = query.shape[0]
    n = pos.shape[0]
    blk_r = min(m, 256)
    grid = m // blk_r
    return pl.pallas_call(
        functools.partial(_knn_body, n=n, k=k, exclude_self=exclude_self,
                          blk_r=blk_r),
        grid=(grid,),
        in_specs=[
            pl.BlockSpec((blk_r, 3), lambda i: (i, 0)),
            pl.BlockSpec((n, 3), lambda i: (0, 0)),
        ],
        out_specs=pl.BlockSpec((blk_r, k), lambda i: (i, 0)),
        out_shape=jax.ShapeDtypeStruct((m, k), jnp.int32),
    )(query, pos)


def _fps_body(xyz_ref, o_ref, d_ref, *, n, m, S):
    X = xyz_ref[0]
    Y = xyz_ref[1]
    Z = xyz_ref[2]
    fiota = (jax.lax.broadcasted_iota(jnp.int32, (S, 128), 0) * 128
             + jax.lax.broadcasted_iota(jnp.int32, (S, 128), 1))
    miota = jax.lax.broadcasted_iota(jnp.int32, (1, m), 1)

    x0 = jnp.sum(jnp.where(fiota == 0, X, 0.0))
    y0 = jnp.sum(jnp.where(fiota == 0, Y, 0.0))
    z0 = jnp.sum(jnp.where(fiota == 0, Z, 0.0))
    dx = X - x0
    dy = Y - y0
    dz = Z - z0
    d_ref[...] = dx * dx + dy * dy + dz * dz
    o_ref[...] = jnp.zeros((1, m), jnp.int32)

    def body(i, _):
        d = d_ref[...]
        mval = jnp.max(d, axis=(0, 1), keepdims=True)
        nxt = jnp.min(jnp.where(d == mval, fiota, n), axis=(0, 1),
                      keepdims=True)
        eq = fiota == nxt
        x = jnp.sum(jnp.where(eq, X, 0.0), axis=(0, 1), keepdims=True)
        y = jnp.sum(jnp.where(eq, Y, 0.0), axis=(0, 1), keepdims=True)
        z = jnp.sum(jnp.where(eq, Z, 0.0), axis=(0, 1), keepdims=True)
        ddx = X - x
        ddy = Y - y
        ddz = Z - z
        dn = ddx * ddx + ddy * ddy + ddz * ddz
        d_ref[...] = jnp.minimum(d, dn)
        o_ref[...] = jnp.where(miota == i, nxt[0], o_ref[...])
        return 0

    jax.lax.fori_loop(1, m, body, 0)


def _fps(pos, m):
    n = pos.shape[0]
    S = n // 128
    xyz = pos.T.reshape(3, S, 128)
    out = pl.pallas_call(
        functools.partial(_fps_body, n=n, m=m, S=S),
        scratch_shapes=[pltpu.VMEM((S, 128), jnp.float32)],
        out_shape=jax.ShapeDtypeStruct((1, m), jnp.int32),
    )(xyz)
    return out[0]


def _sc_gather(table, idx):
    V, D = table.shape
    B = idx.shape[0]
    NW = 32
    b_per_w = B // NW
    chunk = min(b_per_w, 128)
    while 2 * chunk * D * 4 > 400000:
        chunk //= 2
    nchunks = b_per_w // chunk
    mesh = plsc.VectorSubcoreMesh(core_axis_name="c", subcore_axis_name="s")

    @functools.partial(
        pl.kernel, mesh=mesh,
        out_type=jax.ShapeDtypeStruct((B, D), jnp.float32),
        scratch_types=[
            pltpu.VMEM((b_per_w,), jnp.int32),
            pltpu.VMEM((2, chunk, D), jnp.float32),
            pltpu.SemaphoreType.DMA,
            pltpu.SemaphoreType.DMA,
        ],
    )
    def k(table_hbm, idx_hbm, out_hbm, idx_v, rows_v, sem0, sem1):
        wid = lax.axis_index("s") * 2 + lax.axis_index("c")
        base = wid * b_per_w
        sems = (sem0, sem1)
        pltpu.sync_copy(idx_hbm.at[pl.ds(base, b_per_w)], idx_v)
        cps = [None, None]
        cps[0] = pltpu.async_copy(
            table_hbm.at[idx_v.at[pl.ds(0, chunk)]], rows_v.at[0], sem0)
        for c in range(nchunks):
            cb = c % 2
            nb = (c + 1) % 2
            if c + 1 < nchunks:
                cps[nb] = pltpu.async_copy(
                    table_hbm.at[idx_v.at[pl.ds((c + 1) * chunk, chunk)]],
                    rows_v.at[nb], sems[nb])
            cps[cb].wait()
            pltpu.sync_copy(rows_v.at[cb],
                            out_hbm.at[pl.ds(base + c * chunk, chunk)])

    return k(table, idx)


def _bn(gamma, beta, h):
    return jax.nn.relu(gamma * h / jnp.sqrt(1.0 + EPS) + beta)


def _pre_body(*refs, d, P, has_entry):
    if has_entry:
        (x_ref, pos_ref, wi_ref, bi_ref, wsrc_ref, wlin_ref, wdst_ref,
         g_ref, be_ref, w0_ref, b0_ref, t_ref, adst_ref) = refs
    else:
        (x_ref, pos_ref, wi_ref, bi_ref, wsrc_ref, wlin_ref, wdst_ref,
         t_ref, adst_ref) = refs
    x = x_ref[...]
    if has_entry:
        x = _bn(g_ref[...], be_ref[...], x @ w0_ref[...] + b0_ref[...])
    if x.ndim == 3:
        x = jnp.max(x, axis=1)
    d_in = wi_ref.shape[0]
    if x.shape[1] > d_in:
        x = x[:, :d_in]
    xi = jax.nn.relu(x @ wi_ref[...] + bi_ref[...])
    pad = jnp.zeros((x.shape[0], P - (2 * d + 3)), jnp.float32)
    t_ref[...] = jnp.concatenate(
        [xi @ wsrc_ref[...], xi @ wlin_ref[...], pos_ref[...], pad], axis=1)
    adst_ref[...] = xi @ wdst_ref[...]


def _tblock_pre(p, x, pos, entry=None):
    n = pos.shape[0]
    d = p["W_lin"].shape[0]
    P = _pad128(2 * d + 3)
    blk = min(n, 1024)
    grid = n // blk
    if x.ndim == 3:
        x_spec = pl.BlockSpec((blk, x.shape[1], x.shape[2]),
                              lambda i: (i, 0, 0))
    else:
        x_spec = pl.BlockSpec((blk, x.shape[1]), lambda i: (i, 0))
    full = lambda a: pl.BlockSpec(a.shape, lambda i: tuple(0 for _ in a.shape))
    args = [x, pos, p["lin_in"]["W"], p["lin_in"]["b"][None, :],
            p["W_src"], p["W_lin"], p["W_dst"]]
    specs = [x_spec, pl.BlockSpec((blk, 3), lambda i: (i, 0))] + [
        full(a) for a in args[2:]]
    if entry is not None:
        eargs = [entry["gamma"][None, :], entry["beta"][None, :],
                 entry["W"], entry["b"][None, :]]
        args += eargs
        specs += [full(a) for a in eargs]
    T, adst = pl.pallas_call(
        functools.partial(_pre_body, d=d, P=P, has_entry=entry is not None),
        grid=(grid,),
        in_specs=specs,
        out_specs=[pl.BlockSpec((blk, P), lambda i: (i, 0)),
                   pl.BlockSpec((blk, d), lambda i: (i, 0))],
        out_shape=[jax.ShapeDtypeStruct((n, P), jnp.float32),
                   jax.ShapeDtypeStruct((n, d), jnp.float32)],
    )(*args)
    return T, adst


def _conv_body(g_ref, adst_ref, pos_ref,
               pw1_ref, pb1_ref, pw2_ref, pb2_ref,
               aw1_ref, ab1_ref, aw2_ref, ab2_ref,
               wo_ref, bo_ref, wn_ref, bn_ref, gn_ref, ben_ref,
               o_ref, *, d, P, blk, head, n_total):
    e = blk * K
    G = g_ref[...].reshape(e, P)
    a_src = G[:, 0:d]
    h = G[:, d:2 * d]
    psrc = G[:, 2 * d:2 * d + 3]
    pos = pos_ref[...]
    pdst = jnp.broadcast_to(pos[:, None, :], (blk, K, 3)).reshape(e, 3)
    pv = pdst - psrc
    t1 = jax.nn.relu(pv @ pw1_ref[...] + pb1_ref[...])
    delta = jax.nn.relu(t1 @ pw2_ref[...] + pb2_ref[...])
    adst = adst_ref[...]
    ain = (jnp.broadcast_to(adst[:, None, :], (blk, K, d)).reshape(e, d)
           - a_src + delta)
    t2 = jax.nn.relu(ain @ aw1_ref[...] + ab1_ref[...])
    alpha = jax.nn.relu(t2 @ aw2_ref[...] + ab2_ref[...])
    a3 = alpha.reshape(blk, K, d)
    mx = jnp.max(a3, axis=1, keepdims=True)
    ex = jnp.exp(a3 - mx)
    s = jnp.sum(ex, axis=1, keepdims=True)
    al = ex / (s + 1e-16)
    msg = al * (h + delta).reshape(blk, K, d)
    conv = jnp.sum(msg, axis=1)
    x = jax.nn.relu(conv @ wo_ref[...] + bo_ref[...])
    if head:
        pooled = jnp.sum(x, axis=0, keepdims=True) / n_total
        o1 = jax.nn.relu(pooled @ wn_ref[...] + bn_ref[...])
        o_ref[...] = o1 @ gn_ref[...] + ben_ref[...]
    else:
        h2 = x @ wn_ref[...] + bn_ref[...]
        xm = _bn(gn_ref[...], ben_ref[...], h2)
        dp = o_ref.shape[1]
        if dp > xm.shape[1]:
            xm = jnp.concatenate(
                [xm, jnp.zeros((xm.shape[0], dp - xm.shape[1]), jnp.float32)],
                axis=1)
        o_ref[...] = xm


def _tblock_conv(p, G, adst, pos, nxt):
    n, _, P = G.shape
    d = p["W_lin"].shape[0]
    blk = min(n, 256)
    grid = n // blk
    head = nxt[0] == "head"
    if head:
        wn, bn_, gn, ben = (nxt[1]["W"], nxt[1]["b"][None, :],
                            nxt[2]["W"], nxt[2]["b"][None, :])
        out_shape = jax.ShapeDtypeStruct((1, 10), jnp.float32)
        out_spec = pl.BlockSpec((1, 10), lambda i: (0, 0))
    else:
        td = nxt[1]
        wn, bn_, gn, ben = (td["W"], td["b"][None, :],
                            td["gamma"][None, :], td["beta"][None, :])
        d_next = _pad128(td["W"].shape[1])
        out_shape = jax.ShapeDtypeStruct((n, d_next), jnp.float32)
        out_spec = pl.BlockSpec((blk, d_next), lambda i: (i, 0))
    full = lambda a: pl.BlockSpec(a.shape, lambda i: tuple(0 for _ in a.shape))
    args = [G, adst, pos,
            p["pos_nn"]["l1"]["W"], p["pos_nn"]["l1"]["b"][None, :],
            p["pos_nn"]["l2"]["W"], p["pos_nn"]["l2"]["b"][None, :],
            p["attn_nn"]["l1"]["W"], p["attn_nn"]["l1"]["b"][None, :],
            p["attn_nn"]["l2"]["W"], p["attn_nn"]["l2"]["b"][None, :],
            p["lin_out"]["W"], p["lin_out"]["b"][None, :],
            wn, bn_, gn, ben]
    specs = [pl.BlockSpec((blk, K, P), lambda i: (i, 0, 0)),
             pl.BlockSpec((blk, d), lambda i: (i, 0)),
             pl.BlockSpec((blk, 3), lambda i: (i, 0))] + [
        full(a) for a in args[3:]]
    return pl.pallas_call(
        functools.partial(_conv_body, d=d, P=P, blk=blk, head=head,
                          n_total=n),
        grid=(grid,),
        in_specs=specs,
        out_specs=out_spec,
        out_shape=out_shape,
    )(*args)


def kernel(x, pos, batch, params):
    idx = _knn_idx(pos, pos, K, True)
    T, adst = _tblock_pre(params["t_in"], x, pos, entry=params["mlp_input"])
    n = pos.shape[0]
    G = _sc_gather(T, idx.reshape(-1)).reshape(n, K, T.shape[1])
    x = _tblock_conv(params["t_in"], G, adst, pos,
                     ("mlp", params["td"][0]))
    for i in range(4):
        n = pos.shape[0]
        n_sub = n // 4
        idc = _fps(pos, n_sub)
        nbr = _knn_idx(pos[idc], pos, K, False)
        xg = _sc_gather(x, nbr.reshape(-1)).reshape(n_sub, K, x.shape[1])
        pos = pos[idc]
        idx = _knn_idx(pos, pos, K, True)
        p = params["tb"][i]
        T, adst = _tblock_pre(p, xg, pos)
        G = _sc_gather(T, idx.reshape(-1)).reshape(n_sub, K, T.shape[1])
        nxt = (("mlp", params["td"][i + 1]) if i < 3
               else ("head", params["out1"], params["out2"]))
        x = _tblock_conv(p, G, adst, pos, nxt)
    return x


# speedup vs baseline: 9.2993x; 1.0713x over previous
import functools
import jax
import jax.numpy as jnp
from jax import lax
from jax.experimental import pallas as pl
from jax.experimental.pallas import tpu as pltpu
from jax.experimental.pallas import tpu_sc as plsc

EPS = 1e-5
K = 16


def _pad128(v):
    return (v + 127) // 128 * 128


def _knn_body(q_ref, p_ref, o_ref, *, n, k, exclude_self, blk_r):
    r0 = pl.program_id(0) * blk_r
    q = q_ref[...]
    p = p_ref[...]
    qq = jnp.sum(q * q, axis=1)[:, None]
    pp = jnp.sum(p * p, axis=1)[None, :]
    qp = jax.lax.dot_general(q, p, (((1,), (1,)), ((), ())),
                             preferred_element_type=jnp.float32)
    d = qq + pp - 2.0 * qp
    col = jax.lax.broadcasted_iota(jnp.int32, (blk_r, n), 1)
    if exclude_self:
        row = jax.lax.broadcasted_iota(jnp.int32, (blk_r, n), 0) + r0
        d = jnp.where(col == row, d + 1e10, d)
    for j in range(k):
        mv = jnp.min(d, axis=1, keepdims=True)
        idxj = jnp.min(jnp.where(d == mv, col, n), axis=1)
        o_ref[:, j] = idxj
        d = jnp.where(col == idxj[:, None], jnp.float32(jnp.inf), d)


def _knn_idx(query, pos, k, exclude_self):
    m = query.shape[0]
    n = pos.shape[0]
    blk_r = min(m, 512)
    grid = m // blk_r
    return pl.pallas_call(
        functools.partial(_knn_body, n=n, k=k, exclude_self=exclude_self,
                          blk_r=blk_r),
        grid=(grid,),
        in_specs=[
            pl.BlockSpec((blk_r, 3), lambda i: (i, 0)),
            pl.BlockSpec((n, 3), lambda i: (0, 0)),
        ],
        out_specs=pl.BlockSpec((blk_r, k), lambda i: (i, 0)),
        out_shape=jax.ShapeDtypeStruct((m, k), jnp.int32),
    )(query, pos)


def _fps_body(xyz_ref, o_ref, d_ref, *, n, m, S):
    X = xyz_ref[0]
    Y = xyz_ref[1]
    Z = xyz_ref[2]
    fiota = (jax.lax.broadcasted_iota(jnp.int32, (S, 128), 0) * 128
             + jax.lax.broadcasted_iota(jnp.int32, (S, 128), 1))
    miota = jax.lax.broadcasted_iota(jnp.int32, (1, m), 1)

    x0 = jnp.sum(jnp.where(fiota == 0, X, 0.0))
    y0 = jnp.sum(jnp.where(fiota == 0, Y, 0.0))
    z0 = jnp.sum(jnp.where(fiota == 0, Z, 0.0))
    dx = X - x0
    dy = Y - y0
    dz = Z - z0
    d_ref[...] = dx * dx + dy * dy + dz * dz
    o_ref[...] = jnp.zeros((1, m), jnp.int32)

    def body(i, _):
        d = d_ref[...]
        mval = jnp.max(d, axis=(0, 1), keepdims=True)
        nxt = jnp.min(jnp.where(d == mval, fiota, n), axis=(0, 1),
                      keepdims=True)
        eq = fiota == nxt
        x = jnp.sum(jnp.where(eq, X, 0.0), axis=(0, 1), keepdims=True)
        y = jnp.sum(jnp.where(eq, Y, 0.0), axis=(0, 1), keepdims=True)
        z = jnp.sum(jnp.where(eq, Z, 0.0), axis=(0, 1), keepdims=True)
        ddx = X - x
        ddy = Y - y
        ddz = Z - z
        dn = ddx * ddx + ddy * ddy + ddz * ddz
        d_ref[...] = jnp.minimum(d, dn)
        o_ref[...] = jnp.where(miota == i, nxt[0], o_ref[...])
        return 0

    jax.lax.fori_loop(1, m, body, 0)


def _fps(pos, m):
    n = pos.shape[0]
    S = n // 128
    xyz = pos.T.reshape(3, S, 128)
    out = pl.pallas_call(
        functools.partial(_fps_body, n=n, m=m, S=S),
        scratch_shapes=[pltpu.VMEM((S, 128), jnp.float32)],
        out_shape=jax.ShapeDtypeStruct((1, m), jnp.int32),
    )(xyz)
    return out[0]


def _sc_gather(table, idx):
    V, D = table.shape
    B = idx.shape[0]
    NW = 32
    b_per_w = B // NW
    chunk = min(b_per_w, 128)
    while 2 * chunk * D * 4 > 400000:
        chunk //= 2
    nchunks = b_per_w // chunk
    mesh = plsc.VectorSubcoreMesh(core_axis_name="c", subcore_axis_name="s")

    @functools.partial(
        pl.kernel, mesh=mesh,
        out_type=jax.ShapeDtypeStruct((B, D), jnp.float32),
        scratch_types=[
            pltpu.VMEM((b_per_w,), jnp.int32),
            pltpu.VMEM((2, chunk, D), jnp.float32),
            pltpu.SemaphoreType.DMA,
            pltpu.SemaphoreType.DMA,
        ],
    )
    def k(table_hbm, idx_hbm, out_hbm, idx_v, rows_v, sem0, sem1):
        wid = lax.axis_index("s") * 2 + lax.axis_index("c")
        base = wid * b_per_w
        sems = (sem0, sem1)
        pltpu.sync_copy(idx_hbm.at[pl.ds(base, b_per_w)], idx_v)
        cps = [None, None]
        cps[0] = pltpu.async_copy(
            table_hbm.at[idx_v.at[pl.ds(0, chunk)]], rows_v.at[0], sem0)
        for c in range(nchunks):
            cb = c % 2
            nb = (c + 1) % 2
            if c + 1 < nchunks:
                cps[nb] = pltpu.async_copy(
                    table_hbm.at[idx_v.at[pl.ds((c + 1) * chunk, chunk)]],
                    rows_v.at[nb], sems[nb])
            cps[cb].wait()
            pltpu.sync_copy(rows_v.at[cb],
                            out_hbm.at[pl.ds(base + c * chunk, chunk)])

    return k(table, idx)


def _bn(gamma, beta, h):
    return jax.nn.relu(gamma * h / jnp.sqrt(1.0 + EPS) + beta)


def _pre_body(*refs, d, P, has_entry):
    if has_entry:
        (x_ref, pos_ref, wi_ref, bi_ref, wsrc_ref, wlin_ref, wdst_ref,
         g_ref, be_ref, w0_ref, b0_ref, t_ref, adst_ref) = refs
    else:
        (x_ref, pos_ref, wi_ref, bi_ref, wsrc_ref, wlin_ref, wdst_ref,
         t_ref, adst_ref) = refs
    x = x_ref[...]
    if has_entry:
        x = _bn(g_ref[...], be_ref[...], x @ w0_ref[...] + b0_ref[...])
    if x.ndim == 3:
        x = jnp.max(x, axis=1)
    d_in = wi_ref.shape[0]
    if x.shape[1] > d_in:
        x = x[:, :d_in]
    xi = jax.nn.relu(x @ wi_ref[...] + bi_ref[...])
    pad = jnp.zeros((x.shape[0], P - (2 * d + 3)), jnp.float32)
    t_ref[...] = jnp.concatenate(
        [xi @ wsrc_ref[...], xi @ wlin_ref[...], pos_ref[...], pad], axis=1)
    adst_ref[...] = xi @ wdst_ref[...]


def _tblock_pre(p, x, pos, entry=None):
    n = pos.shape[0]
    d = p["W_lin"].shape[0]
    P = _pad128(2 * d + 3)
    blk = min(n, 1024)
    grid = n // blk
    if x.ndim == 3:
        x_spec = pl.BlockSpec((blk, x.shape[1], x.shape[2]),
                              lambda i: (i, 0, 0))
    else:
        x_spec = pl.BlockSpec((blk, x.shape[1]), lambda i: (i, 0))
    full = lambda a: pl.BlockSpec(a.shape, lambda i: tuple(0 for _ in a.shape))
    args = [x, pos, p["lin_in"]["W"], p["lin_in"]["b"][None, :],
            p["W_src"], p["W_lin"], p["W_dst"]]
    specs = [x_spec, pl.BlockSpec((blk, 3), lambda i: (i, 0))] + [
        full(a) for a in args[2:]]
    if entry is not None:
        eargs = [entry["gamma"][None, :], entry["beta"][None, :],
                 entry["W"], entry["b"][None, :]]
        args += eargs
        specs += [full(a) for a in eargs]
    T, adst = pl.pallas_call(
        functools.partial(_pre_body, d=d, P=P, has_entry=entry is not None),
        grid=(grid,),
        in_specs=specs,
        out_specs=[pl.BlockSpec((blk, P), lambda i: (i, 0)),
                   pl.BlockSpec((blk, d), lambda i: (i, 0))],
        out_shape=[jax.ShapeDtypeStruct((n, P), jnp.float32),
                   jax.ShapeDtypeStruct((n, d), jnp.float32)],
    )(*args)
    return T, adst


def _conv_body(g_ref, adst_ref, pos_ref,
               pw1_ref, pb1_ref, pw2_ref, pb2_ref,
               aw1_ref, ab1_ref, aw2_ref, ab2_ref,
               wo_ref, bo_ref, wn_ref, bn_ref, gn_ref, ben_ref,
               o_ref, *, d, P, blk, head, n_total):
    e = blk * K
    G = g_ref[...].reshape(e, P)
    a_src = G[:, 0:d]
    h = G[:, d:2 * d]
    psrc = G[:, 2 * d:2 * d + 3]
    pos = pos_ref[...]
    pdst = jnp.broadcast_to(pos[:, None, :], (blk, K, 3)).reshape(e, 3)
    pv = pdst - psrc
    t1 = jax.nn.relu(pv @ pw1_ref[...] + pb1_ref[...])
    delta = jax.nn.relu(t1 @ pw2_ref[...] + pb2_ref[...])
    adst = adst_ref[...]
    ain = (jnp.broadcast_to(adst[:, None, :], (blk, K, d)).reshape(e, d)
           - a_src + delta)
    t2 = jax.nn.relu(ain @ aw1_ref[...] + ab1_ref[...])
    alpha = jax.nn.relu(t2 @ aw2_ref[...] + ab2_ref[...])
    a3 = alpha.reshape(blk, K, d)
    mx = jnp.max(a3, axis=1, keepdims=True)
    ex = jnp.exp(a3 - mx)
    s = jnp.sum(ex, axis=1, keepdims=True)
    al = ex / (s + 1e-16)
    msg = al * (h + delta).reshape(blk, K, d)
    conv = jnp.sum(msg, axis=1)
    x = jax.nn.relu(conv @ wo_ref[...] + bo_ref[...])
    if head:
        pooled = jnp.sum(x, axis=0, keepdims=True) / n_total
        o1 = jax.nn.relu(pooled @ wn_ref[...] + bn_ref[...])
        o_ref[...] = o1 @ gn_ref[...] + ben_ref[...]
    else:
        h2 = x @ wn_ref[...] + bn_ref[...]
        xm = _bn(gn_ref[...], ben_ref[...], h2)
        dp = o_ref.shape[1]
        if dp > xm.shape[1]:
            xm = jnp.concatenate(
                [xm, jnp.zeros((xm.shape[0], dp - xm.shape[1]), jnp.float32)],
                axis=1)
        o_ref[...] = xm


def _tblock_conv(p, G, adst, pos, nxt):
    n, _, P = G.shape
    d = p["W_lin"].shape[0]
    blk = min(n, 256)
    grid = n // blk
    head = nxt[0] == "head"
    if head:
        wn, bn_, gn, ben = (nxt[1]["W"], nxt[1]["b"][None, :],
                            nxt[2]["W"], nxt[2]["b"][None, :])
        out_shape = jax.ShapeDtypeStruct((1, 10), jnp.float32)
        out_spec = pl.BlockSpec((1, 10), lambda i: (0, 0))
    else:
        td = nxt[1]
        wn, bn_, gn, ben = (td["W"], td["b"][None, :],
                            td["gamma"][None, :], td["beta"][None, :])
        d_next = _pad128(td["W"].shape[1])
        out_shape = jax.ShapeDtypeStruct((n, d_next), jnp.float32)
        out_spec = pl.BlockSpec((blk, d_next), lambda i: (i, 0))
    full = lambda a: pl.BlockSpec(a.shape, lambda i: tuple(0 for _ in a.shape))
    args = [G, adst, pos,
            p["pos_nn"]["l1"]["W"], p["pos_nn"]["l1"]["b"][None, :],
            p["pos_nn"]["l2"]["W"], p["pos_nn"]["l2"]["b"][None, :],
            p["attn_nn"]["l1"]["W"], p["attn_nn"]["l1"]["b"][None, :],
            p["attn_nn"]["l2"]["W"], p["attn_nn"]["l2"]["b"][None, :],
            p["lin_out"]["W"], p["lin_out"]["b"][None, :],
            wn, bn_, gn, ben]
    specs = [pl.BlockSpec((blk, K, P), lambda i: (i, 0, 0)),
             pl.BlockSpec((blk, d), lambda i: (i, 0)),
             pl.BlockSpec((blk, 3), lambda i: (i, 0))] + [
        full(a) for a in args[3:]]
    return pl.pallas_call(
        functools.partial(_conv_body, d=d, P=P, blk=blk, head=head,
                          n_total=n),
        grid=(grid,),
        in_specs=specs,
        out_specs=out_spec,
        out_shape=out_shape,
    )(*args)


def kernel(x, pos, batch, params):
    idx = _knn_idx(pos, pos, K, True)
    T, adst = _tblock_pre(params["t_in"], x, pos, entry=params["mlp_input"])
    n = pos.shape[0]
    G = _sc_gather(T, idx.reshape(-1)).reshape(n, K, T.shape[1])
    x = _tblock_conv(params["t_in"], G, adst, pos,
                     ("mlp", params["td"][0]))
    for i in range(4):
        n = pos.shape[0]
        n_sub = n // 4
        idc = _fps(pos, n_sub)
        nbr = _knn_idx(pos[idc], pos, K, False)
        xg = _sc_gather(x, nbr.reshape(-1)).reshape(n_sub, K, x.shape[1])
        pos = pos[idc]
        idx = _knn_idx(pos, pos, K, True)
        p = params["tb"][i]
        T, adst = _tblock_pre(p, xg, pos)
        G = _sc_gather(T, idx.reshape(-1)).reshape(n_sub, K, T.shape[1])
        nxt = (("mlp", params["td"][i + 1]) if i < 3
               else ("head", params["out1"], params["out2"]))
        x = _tblock_conv(p, G, adst, pos, nxt)
    return x


# speedup vs baseline: 9.3650x; 1.0071x over previous
import functools
import jax
import jax.numpy as jnp
from jax import lax
from jax.experimental import pallas as pl
from jax.experimental.pallas import tpu as pltpu
from jax.experimental.pallas import tpu_sc as plsc

EPS = 1e-5
K = 16


def _pad128(v):
    return (v + 127) // 128 * 128


def _knn_body(q_ref, p_ref, o_ref, *, n, k, exclude_self, blk_r):
    r0 = pl.program_id(0) * blk_r
    q = q_ref[...]
    p = p_ref[...]
    qq = jnp.sum(q * q, axis=1)[:, None]
    pp = jnp.sum(p * p, axis=1)[None, :]
    qp = jax.lax.dot_general(q, p, (((1,), (1,)), ((), ())),
                             preferred_element_type=jnp.float32)
    d = qq + pp - 2.0 * qp
    col = jax.lax.broadcasted_iota(jnp.int32, (blk_r, n), 1)
    if exclude_self:
        row = jax.lax.broadcasted_iota(jnp.int32, (blk_r, n), 0) + r0
        d = jnp.where(col == row, d + 1e10, d)
    for j in range(k):
        mv = jnp.min(d, axis=1, keepdims=True)
        idxj = jnp.min(jnp.where(d == mv, col, n), axis=1)
        o_ref[:, j] = idxj
        d = jnp.where(col == idxj[:, None], jnp.float32(jnp.inf), d)


def _knn_idx(query, pos, k, exclude_self):
    m = query.shape[0]
    n = pos.shape[0]
    blk_r = min(m, 1024)
    grid = m // blk_r
    return pl.pallas_call(
        functools.partial(_knn_body, n=n, k=k, exclude_self=exclude_self,
                          blk_r=blk_r),
        grid=(grid,),
        in_specs=[
            pl.BlockSpec((blk_r, 3), lambda i: (i, 0)),
            pl.BlockSpec((n, 3), lambda i: (0, 0)),
        ],
        out_specs=pl.BlockSpec((blk_r, k), lambda i: (i, 0)),
        out_shape=jax.ShapeDtypeStruct((m, k), jnp.int32),
    )(query, pos)


def _fps_body(xyz_ref, o_ref, d_ref, *, n, m, S):
    X = xyz_ref[0]
    Y = xyz_ref[1]
    Z = xyz_ref[2]
    fiota = (jax.lax.broadcasted_iota(jnp.int32, (S, 128), 0) * 128
             + jax.lax.broadcasted_iota(jnp.int32, (S, 128), 1))
    miota = jax.lax.broadcasted_iota(jnp.int32, (1, m), 1)

    x0 = jnp.sum(jnp.where(fiota == 0, X, 0.0))
    y0 = jnp.sum(jnp.where(fiota == 0, Y, 0.0))
    z0 = jnp.sum(jnp.where(fiota == 0, Z, 0.0))
    dx = X - x0
    dy = Y - y0
    dz = Z - z0
    d_ref[...] = dx * dx + dy * dy + dz * dz
    o_ref[...] = jnp.zeros((1, m), jnp.int32)

    def body(i, _):
        d = d_ref[...]
        mval = jnp.max(d, axis=(0, 1), keepdims=True)
        nxt = jnp.min(jnp.where(d == mval, fiota, n), axis=(0, 1),
                      keepdims=True)
        eq = fiota == nxt
        x = jnp.sum(jnp.where(eq, X, 0.0), axis=(0, 1), keepdims=True)
        y = jnp.sum(jnp.where(eq, Y, 0.0), axis=(0, 1), keepdims=True)
        z = jnp.sum(jnp.where(eq, Z, 0.0), axis=(0, 1), keepdims=True)
        ddx = X - x
        ddy = Y - y
        ddz = Z - z
        dn = ddx * ddx + ddy * ddy + ddz * ddz
        d_ref[...] = jnp.minimum(d, dn)
        o_ref[...] = jnp.where(miota == i, nxt[0], o_ref[...])
        return 0

    jax.lax.fori_loop(1, m, body, 0)


def _fps(pos, m):
    n = pos.shape[0]
    S = n // 128
    xyz = pos.T.reshape(3, S, 128)
    out = pl.pallas_call(
        functools.partial(_fps_body, n=n, m=m, S=S),
        scratch_shapes=[pltpu.VMEM((S, 128), jnp.float32)],
        out_shape=jax.ShapeDtypeStruct((1, m), jnp.int32),
    )(xyz)
    return out[0]


def _sc_gather(table, idx):
    V, D = table.shape
    B = idx.shape[0]
    NW = 32
    b_per_w = B // NW
    chunk = min(b_per_w, 128)
    while 2 * chunk * D * 4 > 400000:
        chunk //= 2
    nchunks = b_per_w // chunk
    mesh = plsc.VectorSubcoreMesh(core_axis_name="c", subcore_axis_name="s")

    @functools.partial(
        pl.kernel, mesh=mesh,
        out_type=jax.ShapeDtypeStruct((B, D), jnp.float32),
        scratch_types=[
            pltpu.VMEM((b_per_w,), jnp.int32),
            pltpu.VMEM((2, chunk, D), jnp.float32),
            pltpu.SemaphoreType.DMA,
            pltpu.SemaphoreType.DMA,
        ],
    )
    def k(table_hbm, idx_hbm, out_hbm, idx_v, rows_v, sem0, sem1):
        wid = lax.axis_index("s") * 2 + lax.axis_index("c")
        base = wid * b_per_w
        sems = (sem0, sem1)
        pltpu.sync_copy(idx_hbm.at[pl.ds(base, b_per_w)], idx_v)
        cps = [None, None]
        cps[0] = pltpu.async_copy(
            table_hbm.at[idx_v.at[pl.ds(0, chunk)]], rows_v.at[0], sem0)
        for c in range(nchunks):
            cb = c % 2
            nb = (c + 1) % 2
            if c + 1 < nchunks:
                cps[nb] = pltpu.async_copy(
                    table_hbm.at[idx_v.at[pl.ds((c + 1) * chunk, chunk)]],
                    rows_v.at[nb], sems[nb])
            cps[cb].wait()
            pltpu.sync_copy(rows_v.at[cb],
                            out_hbm.at[pl.ds(base + c * chunk, chunk)])

    return k(table, idx)


def _bn(gamma, beta, h):
    return jax.nn.relu(gamma * h / jnp.sqrt(1.0 + EPS) + beta)


def _pre_body(*refs, d, P, has_entry):
    if has_entry:
        (x_ref, pos_ref, wi_ref, bi_ref, wsrc_ref, wlin_ref, wdst_ref,
         g_ref, be_ref, w0_ref, b0_ref, t_ref, adst_ref) = refs
    else:
        (x_ref, pos_ref, wi_ref, bi_ref, wsrc_ref, wlin_ref, wdst_ref,
         t_ref, adst_ref) = refs
    x = x_ref[...]
    if has_entry:
        x = _bn(g_ref[...], be_ref[...], x @ w0_ref[...] + b0_ref[...])
    if x.ndim == 3:
        x = jnp.max(x, axis=1)
    d_in = wi_ref.shape[0]
    if x.shape[1] > d_in:
        x = x[:, :d_in]
    xi = jax.nn.relu(x @ wi_ref[...] + bi_ref[...])
    pad = jnp.zeros((x.shape[0], P - (2 * d + 3)), jnp.float32)
    t_ref[...] = jnp.concatenate(
        [xi @ wsrc_ref[...], xi @ wlin_ref[...], pos_ref[...], pad], axis=1)
    adst_ref[...] = xi @ wdst_ref[...]


def _tblock_pre(p, x, pos, entry=None):
    n = pos.shape[0]
    d = p["W_lin"].shape[0]
    P = _pad128(2 * d + 3)
    blk = min(n, 1024)
    grid = n // blk
    if x.ndim == 3:
        x_spec = pl.BlockSpec((blk, x.shape[1], x.shape[2]),
                              lambda i: (i, 0, 0))
    else:
        x_spec = pl.BlockSpec((blk, x.shape[1]), lambda i: (i, 0))
    full = lambda a: pl.BlockSpec(a.shape, lambda i: tuple(0 for _ in a.shape))
    args = [x, pos, p["lin_in"]["W"], p["lin_in"]["b"][None, :],
            p["W_src"], p["W_lin"], p["W_dst"]]
    specs = [x_spec, pl.BlockSpec((blk, 3), lambda i: (i, 0))] + [
        full(a) for a in args[2:]]
    if entry is not None:
        eargs = [entry["gamma"][None, :], entry["beta"][None, :],
                 entry["W"], entry["b"][None, :]]
        args += eargs
        specs += [full(a) for a in eargs]
    T, adst = pl.pallas_call(
        functools.partial(_pre_body, d=d, P=P, has_entry=entry is not None),
        grid=(grid,),
        in_specs=specs,
        out_specs=[pl.BlockSpec((blk, P), lambda i: (i, 0)),
                   pl.BlockSpec((blk, d), lambda i: (i, 0))],
        out_shape=[jax.ShapeDtypeStruct((n, P), jnp.float32),
                   jax.ShapeDtypeStruct((n, d), jnp.float32)],
    )(*args)
    return T, adst


def _conv_body(g_ref, adst_ref, pos_ref,
               pw1_ref, pb1_ref, pw2_ref, pb2_ref,
               aw1_ref, ab1_ref, aw2_ref, ab2_ref,
               wo_ref, bo_ref, wn_ref, bn_ref, gn_ref, ben_ref,
               o_ref, *, d, P, blk, head, n_total):
    e = blk * K
    G = g_ref[...].reshape(e, P)
    a_src = G[:, 0:d]
    h = G[:, d:2 * d]
    psrc = G[:, 2 * d:2 * d + 3]
    pos = pos_ref[...]
    pdst = jnp.broadcast_to(pos[:, None, :], (blk, K, 3)).reshape(e, 3)
    pv = pdst - psrc
    t1 = jax.nn.relu(pv @ pw1_ref[...] + pb1_ref[...])
    delta = jax.nn.relu(t1 @ pw2_ref[...] + pb2_ref[...])
    adst = adst_ref[...]
    ain = (jnp.broadcast_to(adst[:, None, :], (blk, K, d)).reshape(e, d)
           - a_src + delta)
    t2 = jax.nn.relu(ain @ aw1_ref[...] + ab1_ref[...])
    alpha = jax.nn.relu(t2 @ aw2_ref[...] + ab2_ref[...])
    a3 = alpha.reshape(blk, K, d)
    mx = jnp.max(a3, axis=1, keepdims=True)
    ex = jnp.exp(a3 - mx)
    s = jnp.sum(ex, axis=1, keepdims=True)
    al = ex / (s + 1e-16)
    msg = al * (h + delta).reshape(blk, K, d)
    conv = jnp.sum(msg, axis=1)
    x = jax.nn.relu(conv @ wo_ref[...] + bo_ref[...])
    if head:
        pooled = jnp.sum(x, axis=0, keepdims=True) / n_total
        o1 = jax.nn.relu(pooled @ wn_ref[...] + bn_ref[...])
        o_ref[...] = o1 @ gn_ref[...] + ben_ref[...]
    else:
        h2 = x @ wn_ref[...] + bn_ref[...]
        xm = _bn(gn_ref[...], ben_ref[...], h2)
        dp = o_ref.shape[1]
        if dp > xm.shape[1]:
            xm = jnp.concatenate(
                [xm, jnp.zeros((xm.shape[0], dp - xm.shape[1]), jnp.float32)],
                axis=1)
        o_ref[...] = xm


def _tblock_conv(p, G, adst, pos, nxt):
    n, _, P = G.shape
    d = p["W_lin"].shape[0]
    blk = min(n, 256)
    grid = n // blk
    head = nxt[0] == "head"
    if head:
        wn, bn_, gn, ben = (nxt[1]["W"], nxt[1]["b"][None, :],
                            nxt[2]["W"], nxt[2]["b"][None, :])
        out_shape = jax.ShapeDtypeStruct((1, 10), jnp.float32)
        out_spec = pl.BlockSpec((1, 10), lambda i: (0, 0))
    else:
        td = nxt[1]
        wn, bn_, gn, ben = (td["W"], td["b"][None, :],
                            td["gamma"][None, :], td["beta"][None, :])
        d_next = _pad128(td["W"].shape[1])
        out_shape = jax.ShapeDtypeStruct((n, d_next), jnp.float32)
        out_spec = pl.BlockSpec((blk, d_next), lambda i: (i, 0))
    full = lambda a: pl.BlockSpec(a.shape, lambda i: tuple(0 for _ in a.shape))
    args = [G, adst, pos,
            p["pos_nn"]["l1"]["W"], p["pos_nn"]["l1"]["b"][None, :],
            p["pos_nn"]["l2"]["W"], p["pos_nn"]["l2"]["b"][None, :],
            p["attn_nn"]["l1"]["W"], p["attn_nn"]["l1"]["b"][None, :],
            p["attn_nn"]["l2"]["W"], p["attn_nn"]["l2"]["b"][None, :],
            p["lin_out"]["W"], p["lin_out"]["b"][None, :],
            wn, bn_, gn, ben]
    specs = [pl.BlockSpec((blk, K, P), lambda i: (i, 0, 0)),
             pl.BlockSpec((blk, d), lambda i: (i, 0)),
             pl.BlockSpec((blk, 3), lambda i: (i, 0))] + [
        full(a) for a in args[3:]]
    return pl.pallas_call(
        functools.partial(_conv_body, d=d, P=P, blk=blk, head=head,
                          n_total=n),
        grid=(grid,),
        in_specs=specs,
        out_specs=out_spec,
        out_shape=out_shape,
    )(*args)


def kernel(x, pos, batch, params):
    idx = _knn_idx(pos, pos, K, True)
    T, adst = _tblock_pre(params["t_in"], x, pos, entry=params["mlp_input"])
    n = pos.shape[0]
    G = _sc_gather(T, idx.reshape(-1)).reshape(n, K, T.shape[1])
    x = _tblock_conv(params["t_in"], G, adst, pos,
                     ("mlp", params["td"][0]))
    for i in range(4):
        n = pos.shape[0]
        n_sub = n // 4
        idc = _fps(pos, n_sub)
        nbr = _knn_idx(pos[idc], pos, K, False)
        xg = _sc_gather(x, nbr.reshape(-1)).reshape(n_sub, K, x.shape[1])
        pos = pos[idc]
        idx = _knn_idx(pos, pos, K, True)
        p = params["tb"][i]
        T, adst = _tblock_pre(p, xg, pos)
        G = _sc_gather(T, idx.reshape(-1)).reshape(n_sub, K, T.shape[1])
        nxt = (("mlp", params["td"][i + 1]) if i < 3
               else ("head", params["out1"], params["out2"]))
        x = _tblock_conv(p, G, adst, pos, nxt)
    return x


# speedup vs baseline: 9.3664x; 1.0001x over previous
import functools
import jax
import jax.numpy as jnp
from jax import lax
from jax.experimental import pallas as pl
from jax.experimental.pallas import tpu as pltpu
from jax.experimental.pallas import tpu_sc as plsc

EPS = 1e-5
K = 16


def _pad128(v):
    return (v + 127) // 128 * 128


def _knn_body(q_ref, p_ref, o_ref, *, n, k, exclude_self, blk_r):
    r0 = pl.program_id(0) * blk_r
    q = q_ref[...]
    p = p_ref[...]
    qq = jnp.sum(q * q, axis=1)[:, None]
    pp = jnp.sum(p * p, axis=1)[None, :]
    qp = jax.lax.dot_general(q, p, (((1,), (1,)), ((), ())),
                             preferred_element_type=jnp.float32)
    d = qq + pp - 2.0 * qp
    col = jax.lax.broadcasted_iota(jnp.int32, (blk_r, n), 1)
    if exclude_self:
        row = jax.lax.broadcasted_iota(jnp.int32, (blk_r, n), 0) + r0
        d = jnp.where(col == row, d + 1e10, d)
    for j in range(k):
        mv = jnp.min(d, axis=1, keepdims=True)
        idxj = jnp.min(jnp.where(d == mv, col, n), axis=1)
        o_ref[:, j] = idxj
        if j + 1 < k:
            d = jnp.where(col == idxj[:, None], jnp.float32(jnp.inf), d)


def _knn_idx(query, pos, k, exclude_self):
    m = query.shape[0]
    n = pos.shape[0]
    blk_r = min(m, 1024)
    grid = m // blk_r
    return pl.pallas_call(
        functools.partial(_knn_body, n=n, k=k, exclude_self=exclude_self,
                          blk_r=blk_r),
        grid=(grid,),
        in_specs=[
            pl.BlockSpec((blk_r, 3), lambda i: (i, 0)),
            pl.BlockSpec((n, 3), lambda i: (0, 0)),
        ],
        out_specs=pl.BlockSpec((blk_r, k), lambda i: (i, 0)),
        out_shape=jax.ShapeDtypeStruct((m, k), jnp.int32),
    )(query, pos)


def _fps_body(xyz_ref, o_ref, d_ref, *, n, m, S):
    X = xyz_ref[0]
    Y = xyz_ref[1]
    Z = xyz_ref[2]
    fiota = (jax.lax.broadcasted_iota(jnp.int32, (S, 128), 0) * 128
             + jax.lax.broadcasted_iota(jnp.int32, (S, 128), 1))
    miota = jax.lax.broadcasted_iota(jnp.int32, (1, m), 1)

    x0 = jnp.sum(jnp.where(fiota == 0, X, 0.0))
    y0 = jnp.sum(jnp.where(fiota == 0, Y, 0.0))
    z0 = jnp.sum(jnp.where(fiota == 0, Z, 0.0))
    dx = X - x0
    dy = Y - y0
    dz = Z - z0
    d_ref[...] = dx * dx + dy * dy + dz * dz
    o_ref[...] = jnp.zeros((1, m), jnp.int32)

    def body(i, _):
        d = d_ref[...]
        mval = jnp.max(d, axis=(0, 1), keepdims=True)
        nxt = jnp.min(jnp.where(d == mval, fiota, n), axis=(0, 1),
                      keepdims=True)
        eq = fiota == nxt
        x = jnp.sum(jnp.where(eq, X, 0.0), axis=(0, 1), keepdims=True)
        y = jnp.sum(jnp.where(eq, Y, 0.0), axis=(0, 1), keepdims=True)
        z = jnp.sum(jnp.where(eq, Z, 0.0), axis=(0, 1), keepdims=True)
        ddx = X - x
        ddy = Y - y
        ddz = Z - z
        dn = ddx * ddx + ddy * ddy + ddz * ddz
        d_ref[...] = jnp.minimum(d, dn)
        o_ref[...] = jnp.where(miota == i, nxt[0], o_ref[...])
        return 0

    jax.lax.fori_loop(1, m, body, 0)


def _fps(pos, m):
    n = pos.shape[0]
    S = n // 128
    xyz = pos.T.reshape(3, S, 128)
    out = pl.pallas_call(
        functools.partial(_fps_body, n=n, m=m, S=S),
        scratch_shapes=[pltpu.VMEM((S, 128), jnp.float32)],
        out_shape=jax.ShapeDtypeStruct((1, m), jnp.int32),
    )(xyz)
    return out[0]


def _sc_gather(table, idx):
    V, D = table.shape
    B = idx.shape[0]
    NW = 32
    b_per_w = B // NW
    chunk = min(b_per_w, 128)
    while 2 * chunk * D * 4 > 400000:
        chunk //= 2
    nchunks = b_per_w // chunk
    mesh = plsc.VectorSubcoreMesh(core_axis_name="c", subcore_axis_name="s")

    @functools.partial(
        pl.kernel, mesh=mesh,
        out_type=jax.ShapeDtypeStruct((B, D), jnp.float32),
        scratch_types=[
            pltpu.VMEM((b_per_w,), jnp.int32),
            pltpu.VMEM((2, chunk, D), jnp.float32),
            pltpu.SemaphoreType.DMA,
            pltpu.SemaphoreType.DMA,
        ],
    )
    def k(table_hbm, idx_hbm, out_hbm, idx_v, rows_v, sem0, sem1):
        wid = lax.axis_index("s") * 2 + lax.axis_index("c")
        base = wid * b_per_w
        sems = (sem0, sem1)
        pltpu.sync_copy(idx_hbm.at[pl.ds(base, b_per_w)], idx_v)
        cps = [None, None]
        cps[0] = pltpu.async_copy(
            table_hbm.at[idx_v.at[pl.ds(0, chunk)]], rows_v.at[0], sem0)
        for c in range(nchunks):
            cb = c % 2
            nb = (c + 1) % 2
            if c + 1 < nchunks:
                cps[nb] = pltpu.async_copy(
                    table_hbm.at[idx_v.at[pl.ds((c + 1) * chunk, chunk)]],
                    rows_v.at[nb], sems[nb])
            cps[cb].wait()
            pltpu.sync_copy(rows_v.at[cb],
                            out_hbm.at[pl.ds(base + c * chunk, chunk)])

    return k(table, idx)


def _bn(gamma, beta, h):
    return jax.nn.relu(gamma * h / jnp.sqrt(1.0 + EPS) + beta)


def _pre_body(*refs, d, P, has_entry):
    if has_entry:
        (x_ref, pos_ref, wi_ref, bi_ref, wsrc_ref, wlin_ref, wdst_ref,
         g_ref, be_ref, w0_ref, b0_ref, t_ref, adst_ref) = refs
    else:
        (x_ref, pos_ref, wi_ref, bi_ref, wsrc_ref, wlin_ref, wdst_ref,
         t_ref, adst_ref) = refs
    x = x_ref[...]
    if has_entry:
        x = _bn(g_ref[...], be_ref[...], x @ w0_ref[...] + b0_ref[...])
    if x.ndim == 3:
        x = jnp.max(x, axis=1)
    d_in = wi_ref.shape[0]
    if x.shape[1] > d_in:
        x = x[:, :d_in]
    xi = jax.nn.relu(x @ wi_ref[...] + bi_ref[...])
    pad = jnp.zeros((x.shape[0], P - (2 * d + 3)), jnp.float32)
    t_ref[...] = jnp.concatenate(
        [xi @ wsrc_ref[...], xi @ wlin_ref[...], pos_ref[...], pad], axis=1)
    adst_ref[...] = xi @ wdst_ref[...]


def _tblock_pre(p, x, pos, entry=None):
    n = pos.shape[0]
    d = p["W_lin"].shape[0]
    P = _pad128(2 * d + 3)
    blk = min(n, 1024)
    grid = n // blk
    if x.ndim == 3:
        x_spec = pl.BlockSpec((blk, x.shape[1], x.shape[2]),
                              lambda i: (i, 0, 0))
    else:
        x_spec = pl.BlockSpec((blk, x.shape[1]), lambda i: (i, 0))
    full = lambda a: pl.BlockSpec(a.shape, lambda i: tuple(0 for _ in a.shape))
    args = [x, pos, p["lin_in"]["W"], p["lin_in"]["b"][None, :],
            p["W_src"], p["W_lin"], p["W_dst"]]
    specs = [x_spec, pl.BlockSpec((blk, 3), lambda i: (i, 0))] + [
        full(a) for a in args[2:]]
    if entry is not None:
        eargs = [entry["gamma"][None, :], entry["beta"][None, :],
                 entry["W"], entry["b"][None, :]]
        args += eargs
        specs += [full(a) for a in eargs]
    T, adst = pl.pallas_call(
        functools.partial(_pre_body, d=d, P=P, has_entry=entry is not None),
        grid=(grid,),
        in_specs=specs,
        out_specs=[pl.BlockSpec((blk, P), lambda i: (i, 0)),
                   pl.BlockSpec((blk, d), lambda i: (i, 0))],
        out_shape=[jax.ShapeDtypeStruct((n, P), jnp.float32),
                   jax.ShapeDtypeStruct((n, d), jnp.float32)],
    )(*args)
    return T, adst


def _conv_body(g_ref, adst_ref, pos_ref,
               pw1_ref, pb1_ref, pw2_ref, pb2_ref,
               aw1_ref, ab1_ref, aw2_ref, ab2_ref,
               wo_ref, bo_ref, wn_ref, bn_ref, gn_ref, ben_ref,
               o_ref, *, d, P, blk, head, n_total):
    e = blk * K
    G = g_ref[...].reshape(e, P)
    a_src = G[:, 0:d]
    h = G[:, d:2 * d]
    psrc = G[:, 2 * d:2 * d + 3]
    pos = pos_ref[...]
    pdst = jnp.broadcast_to(pos[:, None, :], (blk, K, 3)).reshape(e, 3)
    pv = pdst - psrc
    t1 = jax.nn.relu(pv @ pw1_ref[...] + pb1_ref[...])
    delta = jax.nn.relu(t1 @ pw2_ref[...] + pb2_ref[...])
    adst = adst_ref[...]
    ain = (jnp.broadcast_to(adst[:, None, :], (blk, K, d)).reshape(e, d)
           - a_src + delta)
    t2 = jax.nn.relu(ain @ aw1_ref[...] + ab1_ref[...])
    alpha = jax.nn.relu(t2 @ aw2_ref[...] + ab2_ref[...])
    a3 = alpha.reshape(blk, K, d)
    mx = jnp.max(a3, axis=1, keepdims=True)
    ex = jnp.exp(a3 - mx)
    s = jnp.sum(ex, axis=1, keepdims=True)
    al = ex / (s + 1e-16)
    msg = al * (h + delta).reshape(blk, K, d)
    conv = jnp.sum(msg, axis=1)
    x = jax.nn.relu(conv @ wo_ref[...] + bo_ref[...])
    if head:
        pooled = jnp.sum(x, axis=0, keepdims=True) / n_total
        o1 = jax.nn.relu(pooled @ wn_ref[...] + bn_ref[...])
        o_ref[...] = o1 @ gn_ref[...] + ben_ref[...]
    else:
        h2 = x @ wn_ref[...] + bn_ref[...]
        xm = _bn(gn_ref[...], ben_ref[...], h2)
        dp = o_ref.shape[1]
        if dp > xm.shape[1]:
            xm = jnp.concatenate(
                [xm, jnp.zeros((xm.shape[0], dp - xm.shape[1]), jnp.float32)],
                axis=1)
        o_ref[...] = xm


def _tblock_conv(p, G, adst, pos, nxt):
    n, _, P = G.shape
    d = p["W_lin"].shape[0]
    blk = min(n, 256)
    grid = n // blk
    head = nxt[0] == "head"
    if head:
        wn, bn_, gn, ben = (nxt[1]["W"], nxt[1]["b"][None, :],
                            nxt[2]["W"], nxt[2]["b"][None, :])
        out_shape = jax.ShapeDtypeStruct((1, 10), jnp.float32)
        out_spec = pl.BlockSpec((1, 10), lambda i: (0, 0))
    else:
        td = nxt[1]
        wn, bn_, gn, ben = (td["W"], td["b"][None, :],
                            td["gamma"][None, :], td["beta"][None, :])
        d_next = _pad128(td["W"].shape[1])
        out_shape = jax.ShapeDtypeStruct((n, d_next), jnp.float32)
        out_spec = pl.BlockSpec((blk, d_next), lambda i: (i, 0))
    full = lambda a: pl.BlockSpec(a.shape, lambda i: tuple(0 for _ in a.shape))
    args = [G, adst, pos,
            p["pos_nn"]["l1"]["W"], p["pos_nn"]["l1"]["b"][None, :],
            p["pos_nn"]["l2"]["W"], p["pos_nn"]["l2"]["b"][None, :],
            p["attn_nn"]["l1"]["W"], p["attn_nn"]["l1"]["b"][None, :],
            p["attn_nn"]["l2"]["W"], p["attn_nn"]["l2"]["b"][None, :],
            p["lin_out"]["W"], p["lin_out"]["b"][None, :],
            wn, bn_, gn, ben]
    specs = [pl.BlockSpec((blk, K, P), lambda i: (i, 0, 0)),
             pl.BlockSpec((blk, d), lambda i: (i, 0)),
             pl.BlockSpec((blk, 3), lambda i: (i, 0))] + [
        full(a) for a in args[3:]]
    return pl.pallas_call(
        functools.partial(_conv_body, d=d, P=P, blk=blk, head=head,
                          n_total=n),
        grid=(grid,),
        in_specs=specs,
        out_specs=out_spec,
        out_shape=out_shape,
    )(*args)


def kernel(x, pos, batch, params):
    idx = _knn_idx(pos, pos, K, True)
    T, adst = _tblock_pre(params["t_in"], x, pos, entry=params["mlp_input"])
    n = pos.shape[0]
    G = _sc_gather(T, idx.reshape(-1)).reshape(n, K, T.shape[1])
    x = _tblock_conv(params["t_in"], G, adst, pos,
                     ("mlp", params["td"][0]))
    for i in range(4):
        n = pos.shape[0]
        n_sub = n // 4
        idc = _fps(pos, n_sub)
        nbr = _knn_idx(pos[idc], pos, K, False)
        xg = _sc_gather(x, nbr.reshape(-1)).reshape(n_sub, K, x.shape[1])
        pos = pos[idc]
        idx = _knn_idx(pos, pos, K, True)
        p = params["tb"][i]
        T, adst = _tblock_pre(p, xg, pos)
        G = _sc_gather(T, idx.reshape(-1)).reshape(n_sub, K, T.shape[1])
        nxt = (("mlp", params["td"][i + 1]) if i < 3
               else ("head", params["out1"], params["out2"]))
        x = _tblock_conv(p, G, adst, pos, nxt)
    return x
